# speedup vs baseline: 1.0167x; 1.0167x over previous
import jax
import jax.numpy as jnp
from jax.experimental import pallas as pl
from jax.experimental.pallas import tpu as pltpu

MAX_B = 8
MAX_S = 2048
N_HEADS = 16
HEAD_DIM = 128
Q_LEN = 16


def _body(pos_ref, k_cache_ref, v_cache_ref, k_val_ref, v_val_ref,
          k_out_ref, v_out_ref):
    k_out_ref[...] = k_cache_ref[...]
    v_out_ref[...] = v_cache_ref[...]
    for i in range(Q_LEN):
        p = pos_ref[i]
        k_out_ref[0, 0, pl.ds(p, 1), :] = k_val_ref[0, 0, pl.ds(i, 1), :]
        v_out_ref[0, 0, pl.ds(p, 1), :] = v_val_ref[0, 0, pl.ds(i, 1), :]


def kernel(k_cache, v_cache, input_pos, k_val, v_val):
    cache_spec = pl.BlockSpec((1, 1, MAX_S, HEAD_DIM), lambda b, h: (b, h, 0, 0))
    val_spec = pl.BlockSpec((1, 1, Q_LEN, HEAD_DIM), lambda b, h: (b, h, 0, 0))
    out_shape = jax.ShapeDtypeStruct((MAX_B, N_HEADS, MAX_S, HEAD_DIM), jnp.float32)
    return pl.pallas_call(
        _body,
        grid=(MAX_B, N_HEADS),
        in_specs=[
            pl.BlockSpec(memory_space=pltpu.SMEM),
            cache_spec, cache_spec, val_spec, val_spec,
        ],
        out_specs=[cache_spec, cache_spec],
        out_shape=[out_shape, out_shape],
    )(input_pos, k_cache, v_cache, k_val, v_val)
